# baseline (speedup 1.0000x reference)
import functools

import jax
import jax.numpy as jnp
from jax import lax
from jax.experimental import pallas as pl
from jax.experimental.pallas import tpu as pltpu
from jax.experimental.pallas import tpu_sc as plsc

_NC = 2
_NS = 16
_NW = _NC * _NS
_C = 40
_K = 80
_PER_TILE = _C * _K
_NPAD = _NW * _PER_TILE
_H = 256
_F = 9


def _sc_body(gidx_hbm, emb_hbm, out_hbm, idx_v, rows_v, out_v, sem):
    c = lax.axis_index("c")
    s = lax.axis_index("s")
    wid = s * _NC + c

    def chunk_body(k, carry):
        pltpu.sync_copy(gidx_hbm.at[wid, k], idx_v)
        copies = []
        for i in range(_F):
            copies.append(
                pltpu.async_copy(emb_hbm.at[idx_v.at[i]], rows_v.at[i], sem)
            )
        for cp in copies:
            cp.wait()

        def node_body(j, carry2):
            for col in range(_H // 16):
                sl = pl.ds(col * 16, 16)
                acc = rows_v[0, j, sl]
                for i in range(1, _F):
                    acc = acc + rows_v[i, j, sl]
                out_v[j, sl] = acc
            return carry2

        lax.fori_loop(0, _C, node_body, 0, unroll=False)
        pltpu.sync_copy(out_v, out_hbm.at[pl.ds(wid * _PER_TILE + k * _C, _C)])
        return carry

    lax.fori_loop(0, _K, chunk_body, 0, unroll=False)


def kernel(x, emb):
    n, f = x.shape
    _, v, h = emb.shape
    gidx = x + v * jnp.arange(f, dtype=jnp.int32)[None, :]
    gidx = jnp.zeros((_NPAD, f), jnp.int32).at[:n].set(gidx)
    gidx4 = gidx.reshape(_NW, _K, _C, f).transpose(0, 1, 3, 2)
    emb_flat = emb.reshape(f * v, h)

    mesh = plsc.VectorSubcoreMesh(
        core_axis_name="c", subcore_axis_name="s",
        num_cores=_NC, num_subcores=_NS,
    )
    run = pl.kernel(
        _sc_body,
        out_type=jax.ShapeDtypeStruct((_NPAD, h), jnp.float32),
        mesh=mesh,
        scratch_types=[
            pltpu.VMEM((_F, _C), jnp.int32),
            pltpu.VMEM((_F, _C, h), jnp.float32),
            pltpu.VMEM((_C, h), jnp.float32),
            pltpu.SemaphoreType.DMA,
        ],
    )
    out = run(gidx4, emb_flat)
    return out[:n]


# speedup vs baseline: 1.1290x; 1.1290x over previous
import jax
import jax.numpy as jnp
from jax import lax
from jax.experimental import pallas as pl
from jax.experimental.pallas import tpu as pltpu
from jax.experimental.pallas import tpu_sc as plsc

_NC = 2
_NS = 16
_NW = _NC * _NS
_NPAIR = _NW // 2
_C = 32
_K = 200
_PER_PAIR = _C * _K
_NPAD = _NPAIR * _PER_PAIR
_H = 256
_HH = _H // 2
_F = 9
_ROWS = 900


def _sc_body(gidx_hbm, emb_hbm, out_hbm, table_v, idx_v, out_v, sem):
    c = lax.axis_index("c")
    s = lax.axis_index("s")
    wid = s * _NC + c
    half = wid % 2
    pair = wid // 2

    pltpu.sync_copy(emb_hbm.at[:, pl.ds(half * _HH, _HH)], table_v)

    def chunk_body(k, carry):
        pltpu.sync_copy(gidx_hbm.at[pair, k], idx_v)

        def group_body(g, carry2):
            vecs = [idx_v[i, pl.ds(g * 16, 16)] for i in range(_F)]
            for jj in range(16):
                rows = [vecs[i][jj] for i in range(_F)]
                for colv in range(_HH // 16):
                    sl = pl.ds(colv * 16, 16)
                    acc = table_v[rows[0], sl]
                    for i in range(1, _F):
                        acc = acc + table_v[rows[i], sl]
                    out_v[g * 16 + jj, sl] = acc
            return carry2

        lax.fori_loop(0, _C // 16, group_body, 0, unroll=False)
        pltpu.sync_copy(
            out_v,
            out_hbm.at[pl.ds(pair * _PER_PAIR + k * _C, _C),
                       pl.ds(half * _HH, _HH)],
        )
        return carry

    lax.fori_loop(0, _K, chunk_body, 0, unroll=False)


def kernel(x, emb):
    n, f = x.shape
    _, v, h = emb.shape
    gidx = x + v * jnp.arange(f, dtype=jnp.int32)[None, :]
    gidx = jnp.zeros((_NPAD, f), jnp.int32).at[:n].set(gidx)
    gidx4 = gidx.reshape(_NPAIR, _K, _C, f).transpose(0, 1, 3, 2)
    emb_flat = emb.reshape(f * v, h)

    mesh = plsc.VectorSubcoreMesh(
        core_axis_name="c", subcore_axis_name="s",
        num_cores=_NC, num_subcores=_NS,
    )
    run = pl.kernel(
        _sc_body,
        out_type=jax.ShapeDtypeStruct((_NPAD, h), jnp.float32),
        mesh=mesh,
        scratch_types=[
            pltpu.VMEM((_ROWS, _HH), jnp.float32),
            pltpu.VMEM((_F, _C), jnp.int32),
            pltpu.VMEM((_C, _HH), jnp.float32),
            pltpu.SemaphoreType.DMA,
        ],
    )
    out = run(gidx4, emb_flat)
    return out[:n]
